# baseline (speedup 1.0000x reference)
import functools

import jax
import jax.numpy as jnp
from jax.experimental import pallas as pl
from jax.experimental.pallas import tpu as pltpu

B = 4096
G = 19264
D = 64
NSTREAM = 4
BQ = 64
BM = NSTREAM * BQ


GMAIN = 19200
GTAIL = G - GMAIN


def _dot(a, b):
    return jax.lax.dot_general(
        a, b, dimension_numbers=(((1,), (0,)), ((), ())),
        preferred_element_type=jnp.float32,
    )


def _matmul_body(xa_ref, xb_ref, emb_ref, out_ref):
    emb = emb_ref[...]
    out_ref[...] = (_dot(xa_ref[...], emb[:GMAIN, :])
                    + _dot(xb_ref[...], emb[GMAIN:, :]))


@functools.partial(jax.jit, static_argnames=())
def _feat(x, emb):
    x_tail = jax.lax.slice(x, (0, GMAIN), (B, G))
    grid = (B // BM,)
    return pl.pallas_call(
        _matmul_body,
        grid=grid,
        in_specs=[
            pl.BlockSpec((BM, GMAIN), lambda i: (i, 0)),
            pl.BlockSpec((BM, GTAIL), lambda i: (i, 0)),
            pl.BlockSpec((G, D), lambda i: (0, 0)),
        ],
        out_specs=pl.BlockSpec((BM, D), lambda i: (i, 0)),
        out_shape=jax.ShapeDtypeStruct((B, D), jnp.float32),
    )(x, x_tail, emb)


def kernel(x, emb):
    feat = _feat(x, emb)
    return (feat, emb)
